# baseline (speedup 1.0000x reference)
import functools

import jax
import jax.numpy as jnp
from jax import lax
from jax.experimental import pallas as pl
from jax.experimental.pallas import tpu as pltpu
from jax.experimental.pallas import tpu_sc as plsc

_DN_T = (((1,), (1,)), ((), ()))

_NEG = -3.0e38


def _gate_body(x_ref, Wg_ref, bg_ref, glT_ref):
    glT_ref[...] = lax.dot_general(
        Wg_ref[...], x_ref[...], _DN_T, preferred_element_type=jnp.float32
    ) + bg_ref[...][:, None]


def _route_body(E, TPW, glT_hbm, fullT_hbm, sparseT_hbm, gl_v, full_v, sparse_v):
    wid = lax.axis_index("s") * 2 + lax.axis_index("c")
    base = wid * TPW
    pltpu.sync_copy(glT_hbm.at[:, pl.ds(base, TPW)], gl_v)
    for c in range(TPW // 16):
        sl = pl.ds(c * 16, 16)
        p = [gl_v[e, sl] for e in range(E)]
        m = p[0]
        for e in range(1, E):
            m = jnp.maximum(m, p[e])
        eg = [jnp.exp(p[e] - m) for e in range(E)]
        ssum = eg[0]
        for e in range(1, E):
            ssum = ssum + eg[e]
        fp = [eg[e] / ssum for e in range(E)]
        for e in range(E):
            full_v[e, sl] = fp[e]
        v1 = p[0]
        for e in range(1, E):
            v1 = jnp.maximum(v1, p[e])
        i1 = jnp.full((16,), E, jnp.int32)
        for e in range(E - 1, -1, -1):
            i1 = jnp.where(p[e] == v1, e, i1)
        p2 = [jnp.where(i1 == e, _NEG, p[e]) for e in range(E)]
        v2 = p2[0]
        for e in range(1, E):
            v2 = jnp.maximum(v2, p2[e])
        i2 = jnp.full((16,), E, jnp.int32)
        for e in range(E - 1, -1, -1):
            i2 = jnp.where(p2[e] == v2, e, i2)
        pv1 = fp[0]
        pv2 = jnp.where(i1 == 0, _NEG, fp[0])
        for e in range(1, E):
            pv1 = jnp.maximum(pv1, fp[e])
            pv2 = jnp.maximum(pv2, jnp.where(i1 == e, _NEG, fp[e]))
        s = pv1 + pv2
        w1 = pv1 / s
        w2 = pv2 / s
        for e in range(E):
            sparse_v[e, sl] = jnp.where(
                i1 == e, w1, jnp.where(i2 == e, w2, 0.0))
    pltpu.sync_copy(full_v, fullT_hbm.at[:, pl.ds(base, TPW)])
    pltpu.sync_copy(sparse_v, sparseT_hbm.at[:, pl.ds(base, TPW)])


def _moe_body(x_ref, Wef_ref, bef_ref, Wc_ref, bc_ref, spT_ref,
              logitsT_ref, mixed_ref):
    E = spT_ref.shape[0]
    BT = x_ref.shape[0]
    H = Wef_ref.shape[0] // E
    xt = x_ref[...]
    sp = spT_ref[...].T

    acc_a = jnp.zeros((BT, H), jnp.float32)
    acc_b = jnp.zeros((BT, H), jnp.float32)
    for e in range(0, E, 2):
        ha = lax.dot_general(xt, Wef_ref[pl.ds(e * H, H)], _DN_T,
                             preferred_element_type=jnp.float32)
        hb = lax.dot_general(xt, Wef_ref[pl.ds((e + 1) * H, H)], _DN_T,
                             preferred_element_type=jnp.float32)
        ha = jnp.maximum(ha + bef_ref[pl.ds(e * H, H)][None, :], 0.0)
        hb = jnp.maximum(hb + bef_ref[pl.ds((e + 1) * H, H)][None, :], 0.0)
        acc_a = acc_a + sp[:, e:e + 1] * ha
        acc_b = acc_b + sp[:, e + 1:e + 2] * hb
    acc = acc_a + acc_b
    mixed_ref[...] = acc

    logitsT_ref[...] = lax.dot_general(
        Wc_ref[...], acc, _DN_T, preferred_element_type=jnp.float32
    ) + bc_ref[...][:, None]


def kernel(x, Wg, bg, We, be, Wc, bc):
    B, D = x.shape
    E, H, _ = We.shape
    C = Wc.shape[0]
    Wef = We.reshape(E * H, D)
    bef = be.reshape(E * H)

    BT = 1024 if B % 1024 == 0 else B
    grid = (B // BT,)

    glT = pl.pallas_call(
        _gate_body,
        grid=grid,
        in_specs=[
            pl.BlockSpec((BT, D), lambda i: (i, 0)),
            pl.BlockSpec((E, D), lambda i: (0, 0)),
            pl.BlockSpec((E,), lambda i: (0,)),
        ],
        out_specs=pl.BlockSpec((E, BT), lambda i: (0, i)),
        out_shape=jax.ShapeDtypeStruct((E, B), jnp.float32),
    )(x, Wg, bg)

    info = plsc.get_sparse_core_info()
    NW = info.num_cores * info.num_subcores
    TPW = B // NW
    mesh = plsc.VectorSubcoreMesh(core_axis_name="c", subcore_axis_name="s")
    fullT, sparseT = pl.kernel(
        functools.partial(_route_body, E, TPW),
        mesh=mesh,
        out_type=[
            jax.ShapeDtypeStruct((E, B), jnp.float32),
            jax.ShapeDtypeStruct((E, B), jnp.float32),
        ],
        scratch_types=[
            pltpu.VMEM((E, TPW), jnp.float32),
            pltpu.VMEM((E, TPW), jnp.float32),
            pltpu.VMEM((E, TPW), jnp.float32),
        ],
    )(glT)

    logitsT, mixed = pl.pallas_call(
        _moe_body,
        grid=grid,
        in_specs=[
            pl.BlockSpec((BT, D), lambda i: (i, 0)),
            pl.BlockSpec((E * H, D), lambda i: (0, 0)),
            pl.BlockSpec((E * H,), lambda i: (0,)),
            pl.BlockSpec((C, H), lambda i: (0, 0)),
            pl.BlockSpec((C,), lambda i: (0,)),
            pl.BlockSpec((E, BT), lambda i: (0, i)),
        ],
        out_specs=[
            pl.BlockSpec((C, BT), lambda i: (0, i)),
            pl.BlockSpec((BT, H), lambda i: (i, 0)),
        ],
        out_shape=[
            jax.ShapeDtypeStruct((C, B), jnp.float32),
            jax.ShapeDtypeStruct((B, H), jnp.float32),
        ],
    )(x, Wef, bef, Wc, bc, sparseT)

    return (logitsT.T, sparseT.T, mixed, fullT.T)
